# Initial kernel scaffold; baseline (speedup 1.0000x reference)
import jax
import jax.numpy as jnp
from jax.experimental import pallas as pl


def kernel(x, edge_index, W_gcn, b_gcn, W_bil, b_bil, perm):
    raise NotImplementedError("write your pallas kernel here")


# speedup vs baseline: 11.2047x; 11.2047x over previous
import functools

import jax
import jax.numpy as jnp
from jax import lax
from jax.experimental import pallas as pl
from jax.experimental.pallas import tpu as pltpu
from jax.experimental.pallas import tpu_sc as plsc

N = 10000
E = 320000
F = 128
NP = 10240
NC, NS = 2, 16
NW = NC * NS
ROWS_PER_TILE = NP // NS
CE = 128
CPT = 80
EP = NW * CPT * CE
DUMMY = NP - 8
RB = 1024

_mesh = plsc.VectorSubcoreMesh(core_axis_name="c", subcore_axis_name="s")


DCH = 1024
EPW = EP // NW


HR = NP // 16
HRT = HR // NS


@functools.partial(
    pl.kernel,
    out_type=jax.ShapeDtypeStruct((NC, HR, 16), jnp.float32),
    mesh=_mesh,
    compiler_params=pltpu.CompilerParams(needs_layout_passes=False, use_tc_tiling_on_sc=False),
    scratch_types=[
        pltpu.VMEM((HR, 16), jnp.float32),
        pltpu.VMEM((1, DCH), jnp.int32),
        pltpu.VMEM_SHARED((NS, HR, 16), jnp.float32),
        pltpu.VMEM((NS, HRT, 16), jnp.float32),
        pltpu.VMEM((HRT, 16), jnp.float32),
    ],
)
def _deg_kernel(dstp_hbm, zeros_hbm, hist_out, lhist, dbuf, stage, rbuf, obuf):
    c = lax.axis_index("c")
    s = lax.axis_index("s")
    w = s * NC + c
    pltpu.sync_copy(zeros_hbm, lhist)
    base = w * EPW
    ones = jnp.full((16,), 1.0, jnp.float32)

    def chunk_body(i, _):
        off = pl.multiple_of(base + i * DCH, DCH)
        pltpu.sync_copy(dstp_hbm.at[pl.ds(off, DCH)], dbuf.at[0])

        def inner(k, _):
            idx = dbuf[0, pl.ds(pl.multiple_of(k * 16, 16), 16)]
            plsc.addupdate_scatter(lhist, [idx >> 4, idx & 15], ones)
            return 0

        lax.fori_loop(0, DCH // 16, inner, 0)
        return 0

    lax.fori_loop(0, EPW // DCH, chunk_body, 0)
    pltpu.sync_copy(lhist, stage.at[s])
    plsc.subcore_barrier()
    row0 = s * HRT
    pltpu.sync_copy(stage.at[:, pl.ds(row0, HRT)], rbuf)

    def red(k, _):
        acc = rbuf[0, k]
        for r in range(1, NS):
            acc = acc + rbuf[r, k]
        obuf[k] = acc
        return 0

    lax.fori_loop(0, HRT, red, 0)
    pltpu.sync_copy(obuf, hist_out.at[c, pl.ds(row0, HRT)])


def _enc_body(x_ref, w_ref, histT_ref, g_ref, dinv_ref):
    deg = histT_ref[:, 0:1] + histT_ref[:, 1:2] + 1.0
    dinv = lax.rsqrt(jnp.maximum(deg, 1.0))
    h = jnp.dot(x_ref[...], w_ref[...], preferred_element_type=jnp.float32)
    g_ref[...] = h * dinv
    dinv_ref[...] = dinv


def _encode(x_pad, W_gcn, histT):
    return pl.pallas_call(
        _enc_body,
        grid=(NP // RB,),
        in_specs=[
            pl.BlockSpec((RB, F), lambda i: (i, 0)),
            pl.BlockSpec((F, F), lambda i: (0, 0)),
            pl.BlockSpec((RB, NC), lambda i: (i, 0)),
        ],
        out_specs=[
            pl.BlockSpec((RB, F), lambda i: (i, 0)),
            pl.BlockSpec((RB, 1), lambda i: (i, 0)),
        ],
        out_shape=[
            jax.ShapeDtypeStruct((NP, F), jnp.float32),
            jax.ShapeDtypeStruct((NP, 1), jnp.float32),
        ],
    )(x_pad, W_gcn, histT)


@functools.partial(
    pl.kernel,
    out_type=jax.ShapeDtypeStruct((NC, NP, F), jnp.float32),
    mesh=_mesh,
    compiler_params=pltpu.CompilerParams(needs_layout_passes=False, use_tc_tiling_on_sc=False),
    scratch_types=[
        pltpu.VMEM_SHARED((NP, F), jnp.float32),
        pltpu.VMEM((2, CE), jnp.int32),
        pltpu.VMEM((2, CE), jnp.int32),
        pltpu.VMEM((2, CE, F), jnp.float32),
        pltpu.SemaphoreType.DMA,
        pltpu.SemaphoreType.DMA,
    ],
)
def _edge_kernel(g_hbm, srcp_hbm, dstp_hbm, zeros2d_hbm, s_out,
                 acc, sbuf, dbuf, rbuf, sem0, sem1):
    c = lax.axis_index("c")
    s = lax.axis_index("s")
    w = c * NS + s
    row0 = s * ROWS_PER_TILE
    pltpu.sync_copy(zeros2d_hbm, acc.at[pl.ds(row0, ROWS_PER_TILE)])
    plsc.subcore_barrier()

    tbase = w * CPT * CE
    sems = (sem0, sem1)

    def start(i, b):
        off = pl.multiple_of(tbase + i * CE, CE)
        pltpu.sync_copy(srcp_hbm.at[pl.ds(off, CE)], sbuf.at[b])
        pltpu.sync_copy(dstp_hbm.at[pl.ds(off, CE)], dbuf.at[b])
        pltpu.async_copy(g_hbm.at[sbuf.at[b]], rbuf.at[b], sems[b])

    start(0, 0)

    def outer(j, _):
        for b in range(2):
            i = j * 2 + b

            @pl.when(i + 1 < CPT)
            def _():
                start(i + 1, 1 - b)

            pltpu.make_async_copy(g_hbm.at[sbuf.at[b]], rbuf.at[b], sems[b]).wait()
            pltpu.sync_copy(rbuf.at[b], acc.at[dbuf.at[b]], add=True)
        return 0

    lax.fori_loop(0, CPT // 2, outer, 0)
    plsc.subcore_barrier()
    pltpu.sync_copy(acc.at[pl.ds(row0, ROWS_PER_TILE)],
                    s_out.at[c, pl.ds(row0, ROWS_PER_TILE)])


def _zt_body(s0_ref, s1_ref, g_ref, dinv_ref, bg_ref, wb_ref, z_ref, t_ref):
    z = dinv_ref[...] * (s0_ref[...] + s1_ref[...] + g_ref[...]) + bg_ref[...]
    z_ref[...] = z
    t_ref[...] = jnp.dot(z, wb_ref[...], preferred_element_type=jnp.float32)


def _zt(S0, S1, g, dinv, bg2d, Wb):
    return pl.pallas_call(
        _zt_body,
        grid=(NP // RB,),
        in_specs=[
            pl.BlockSpec((RB, F), lambda i: (i, 0)),
            pl.BlockSpec((RB, F), lambda i: (i, 0)),
            pl.BlockSpec((RB, F), lambda i: (i, 0)),
            pl.BlockSpec((RB, 1), lambda i: (i, 0)),
            pl.BlockSpec((1, F), lambda i: (0, 0)),
            pl.BlockSpec((F, F), lambda i: (0, 0)),
        ],
        out_specs=[
            pl.BlockSpec((RB, F), lambda i: (i, 0)),
            pl.BlockSpec((RB, F), lambda i: (i, 0)),
        ],
        out_shape=[
            jax.ShapeDtypeStruct((NP, F), jnp.float32),
            jax.ShapeDtypeStruct((NP, F), jnp.float32),
        ],
    )(S0, S1, g, dinv, bg2d, Wb)


RPW = NP // NW
PK = 64


@functools.partial(
    pl.kernel,
    out_type=jax.ShapeDtypeStruct((NP, F), jnp.float32),
    mesh=_mesh,
    compiler_params=pltpu.CompilerParams(needs_layout_passes=False, use_tc_tiling_on_sc=False),
    scratch_types=[
        pltpu.VMEM((RPW,), jnp.int32),
        pltpu.VMEM((2, PK, F), jnp.float32),
        pltpu.SemaphoreType.DMA,
        pltpu.SemaphoreType.DMA,
    ],
)
def _perm_kernel(z_hbm, permp_hbm, zp_out, idxv, rbuf, sem0, sem1):
    c = lax.axis_index("c")
    s = lax.axis_index("s")
    w = c * NS + s
    base = w * RPW
    pltpu.sync_copy(permp_hbm.at[pl.ds(base, RPW)], idxv)
    sems = (sem0, sem1)

    def start(j, b):
        pltpu.async_copy(z_hbm.at[idxv.at[pl.ds(j * PK, PK)]], rbuf.at[b], sems[b])

    start(0, 0)
    for j in range(RPW // PK):
        b = j % 2
        if j + 1 < RPW // PK:
            start(j + 1, 1 - b)
        pltpu.make_async_copy(z_hbm.at[idxv.at[pl.ds(j * PK, PK)]],
                              rbuf.at[b], sems[b]).wait()
        pltpu.sync_copy(rbuf.at[b], zp_out.at[pl.ds(base + j * PK, PK)])


def _score_body(z_ref, t_ref, zp_ref, bb_ref, pos_ref, neg_ref):
    t = t_ref[...]
    bb = bb_ref[0, 0]
    pos_ref[...] = jnp.sum(t * z_ref[...], axis=1, keepdims=True) + bb
    neg_ref[...] = jnp.sum(t * zp_ref[...], axis=1, keepdims=True) + bb


def _scores(z, t, zp, bb2d):
    return pl.pallas_call(
        _score_body,
        grid=(NP // RB,),
        in_specs=[
            pl.BlockSpec((RB, F), lambda i: (i, 0)),
            pl.BlockSpec((RB, F), lambda i: (i, 0)),
            pl.BlockSpec((RB, F), lambda i: (i, 0)),
            pl.BlockSpec((1, 1), lambda i: (0, 0)),
        ],
        out_specs=[
            pl.BlockSpec((RB, 1), lambda i: (i, 0)),
            pl.BlockSpec((RB, 1), lambda i: (i, 0)),
        ],
        out_shape=[
            jax.ShapeDtypeStruct((NP, 1), jnp.float32),
            jax.ShapeDtypeStruct((NP, 1), jnp.float32),
        ],
    )(z, t, zp, bb2d)


def kernel(x, edge_index, W_gcn, b_gcn, W_bil, b_bil, perm):
    src = edge_index[0].astype(jnp.int32)
    dst = edge_index[1].astype(jnp.int32)
    srcp = jnp.concatenate([src, jnp.zeros((EP - E,), jnp.int32)])
    dstp = jnp.concatenate([dst, jnp.full((EP - E,), DUMMY, jnp.int32)])
    x_pad = jnp.pad(x, ((0, NP - N), (0, 0)))
    permp = jnp.concatenate([perm.astype(jnp.int32),
                             jnp.zeros((NP - N,), jnp.int32)])
    zeros1d = jnp.zeros((HR, 16), jnp.float32)
    zeros2d = jnp.zeros((ROWS_PER_TILE, F), jnp.float32)

    hist = _deg_kernel(dstp, zeros1d)
    histT = jnp.transpose(jnp.reshape(hist, (NC, NP)))
    g, dinv = _encode(x_pad, W_gcn, histT)
    S = _edge_kernel(g, srcp, dstp, zeros2d)
    z, t = _zt(S[0], S[1], g, dinv, jnp.reshape(b_gcn, (1, F)),
               jnp.reshape(W_bil, (F, F)))
    zp = _perm_kernel(z, permp)
    pos, neg = _scores(z, t, zp, jnp.reshape(b_bil, (1, 1)))
    return (pos[:N], neg[:N])
